# baseline (speedup 1.0000x reference)
import functools

import jax
import jax.numpy as jnp
from jax.experimental import pallas as pl
from jax.experimental.pallas import tpu as pltpu

N = 10000
BI = 200
BX = 2000


def _gcn_body(adj_ref, x_ref, w1_ref, b1_ref, w2_ref, b2_ref, out_ref,
              s1_ref, s2_ref):
    p = pl.program_id(0)
    i = pl.program_id(1)

    @pl.when((p == 0) & (i == 0))
    def _compute_s1():
        s1_ref[...] = jnp.dot(x_ref[...], w1_ref[...],
                              preferred_element_type=jnp.float32)

    @pl.when(p == 0)
    def _layer1():
        h = jnp.dot(adj_ref[...], s1_ref[...],
                    preferred_element_type=jnp.float32) + b1_ref[...]
        h = jnp.maximum(h, 0.0)
        s2_ref[pl.ds(i * BI, BI), :] = jnp.dot(
            h, w2_ref[...], preferred_element_type=jnp.float32)

    @pl.when(p == 1)
    def _layer2():
        out_ref[...] = jnp.dot(adj_ref[...], s2_ref[...],
                               preferred_element_type=jnp.float32) + b2_ref[...]


@functools.partial(jax.jit, static_argnames=("interpret",))
def _gcn(x, adj, W1, b1, W2, b2, interpret=False):
    nfeat = x.shape[1]
    nhid = W1.shape[1]
    nclass = W2.shape[1]

    return pl.pallas_call(
        _gcn_body,
        grid=(2, N // BI),
        in_specs=[
            pl.BlockSpec(
                (BI, N),
                lambda p, i: (jnp.where(p == 0, i, N // BI - 1 - i), 0)),
            pl.BlockSpec((N, nfeat), lambda p, i: (0, 0)),
            pl.BlockSpec((nfeat, nhid), lambda p, i: (0, 0)),
            pl.BlockSpec((1, nhid), lambda p, i: (0, 0)),
            pl.BlockSpec((nhid, nclass), lambda p, i: (0, 0)),
            pl.BlockSpec((1, nclass), lambda p, i: (0, 0)),
        ],
        out_specs=pl.BlockSpec(
            (BI, nclass),
            lambda p, i: (jnp.where(p == 1, N // BI - 1 - i, 0), 0)),
        out_shape=jax.ShapeDtypeStruct((N, nclass), jnp.float32),
        scratch_shapes=[
            pltpu.VMEM((N, nhid), jnp.float32),
            pltpu.VMEM((N, nclass), jnp.float32),
        ],
        interpret=interpret,
    )(adj, x, W1, b1.reshape(1, -1), W2, b2.reshape(1, -1))


def kernel(x, adj, W1, b1, W2, b2):
    return _gcn(x, adj, W1, b1, W2, b2)


# speedup vs baseline: 1.0269x; 1.0269x over previous
import functools

import jax
import jax.numpy as jnp
from jax.experimental import pallas as pl
from jax.experimental.pallas import tpu as pltpu

N = 10000
BI = 400


def _gcn_body(adj_ref, x_ref, w1_ref, b1_ref, w2_ref, b2_ref, out_ref,
              s1_ref, s2_ref):
    p = pl.program_id(0)
    i = pl.program_id(1)

    @pl.when((p == 0) & (i == 0))
    def _compute_s1():
        s1_ref[...] = jnp.dot(x_ref[...], w1_ref[...],
                              preferred_element_type=jnp.float32)

    @pl.when(p == 0)
    def _layer1():
        h = jnp.dot(adj_ref[...], s1_ref[...],
                    preferred_element_type=jnp.float32) + b1_ref[...]
        h = jnp.maximum(h, 0.0)
        s2_ref[pl.ds(i * BI, BI), :] = jnp.dot(
            h, w2_ref[...], preferred_element_type=jnp.float32)

    @pl.when(p == 1)
    def _layer2():
        out_ref[...] = jnp.dot(adj_ref[...], s2_ref[...],
                               preferred_element_type=jnp.float32) + b2_ref[...]


@functools.partial(jax.jit, static_argnames=("interpret",))
def _gcn(x, adj, W1, b1, W2, b2, interpret=False):
    nfeat = x.shape[1]
    nhid = W1.shape[1]
    nclass = W2.shape[1]

    return pl.pallas_call(
        _gcn_body,
        grid=(2, N // BI),
        in_specs=[
            pl.BlockSpec(
                (BI, N),
                lambda p, i: (jnp.where(p == 0, i, N // BI - 1 - i), 0)),
            pl.BlockSpec((N, nfeat), lambda p, i: (0, 0)),
            pl.BlockSpec((nfeat, nhid), lambda p, i: (0, 0)),
            pl.BlockSpec((1, nhid), lambda p, i: (0, 0)),
            pl.BlockSpec((nhid, nclass), lambda p, i: (0, 0)),
            pl.BlockSpec((1, nclass), lambda p, i: (0, 0)),
        ],
        out_specs=pl.BlockSpec(
            (BI, nclass),
            lambda p, i: (jnp.where(p == 1, N // BI - 1 - i, 0), 0)),
        out_shape=jax.ShapeDtypeStruct((N, nclass), jnp.float32),
        scratch_shapes=[
            pltpu.VMEM((N, nhid), jnp.float32),
            pltpu.VMEM((N, nclass), jnp.float32),
        ],
        interpret=interpret,
    )(adj, x, W1, b1.reshape(1, -1), W2, b2.reshape(1, -1))


def kernel(x, adj, W1, b1, W2, b2):
    return _gcn(x, adj, W1, b1, W2, b2)
